# initial kernel scaffold (unmeasured)
import functools

import jax
import jax.numpy as jnp
from jax import lax
from jax.experimental import pallas as pl
from jax.experimental.pallas import tpu as pltpu

N_DEV = 16
BF16 = jnp.bfloat16


def _barrier(my):
    bsem = pltpu.get_barrier_semaphore()
    for j in range(N_DEV):
        @pl.when(my != j)
        def _(j=j):
            pl.semaphore_signal(
                bsem, inc=1, device_id=(j,),
                device_id_type=pl.DeviceIdType.MESH,
            )
    pl.semaphore_wait(bsem, N_DEV - 1)


def _a2a_reduce(kv_chunks, own_chunk):
    n, two, B, S, C = kv_chunks.shape

    def body(kv_ref, own_ref, out_ref, recv_ref, send_sems, recv_sems):
        my = lax.axis_index("i")
        _barrier(my)
        rdmas = []
        for d in range(1, N_DEV):
            tgt = lax.rem(my + d, N_DEV)
            rdma = pltpu.make_async_remote_copy(
                src_ref=kv_ref.at[tgt],
                dst_ref=recv_ref.at[d - 1],
                send_sem=send_sems.at[d - 1],
                recv_sem=recv_sems.at[d - 1],
                device_id=(tgt,),
                device_id_type=pl.DeviceIdType.MESH,
            )
            rdma.start()
            rdmas.append(rdma)
        acc = own_ref[...].astype(jnp.float32)
        for d in range(1, N_DEV):
            rdmas[d - 1].wait_recv()
            acc = acc + recv_ref[d - 1].astype(jnp.float32)
        for d in range(1, N_DEV):
            rdmas[d - 1].wait_send()
        out_ref[...] = acc.astype(BF16)

    return pl.pallas_call(
        body,
        out_shape=jax.ShapeDtypeStruct((two, B, S, C), BF16),
        in_specs=[
            pl.BlockSpec(memory_space=pltpu.ANY),
            pl.BlockSpec(memory_space=pltpu.VMEM),
        ],
        out_specs=pl.BlockSpec(memory_space=pltpu.VMEM),
        scratch_shapes=[
            pltpu.VMEM((N_DEV - 1, two, B, S, C), BF16),
            pltpu.SemaphoreType.DMA((N_DEV - 1,)),
            pltpu.SemaphoreType.DMA((N_DEV - 1,)),
        ],
        compiler_params=pltpu.CompilerParams(collective_id=0),
    )(kv_chunks, own_chunk)


def _allgather_o(o_own):
    B, S, C = o_own.shape

    def body(o_ref, out_ref, recv_ref, send_sems, recv_sems):
        my = lax.axis_index("i")
        _barrier(my)
        rdmas = []
        for d in range(1, N_DEV):
            tgt = lax.rem(my + d, N_DEV)
            rdma = pltpu.make_async_remote_copy(
                src_ref=o_ref,
                dst_ref=recv_ref.at[d - 1],
                send_sem=send_sems.at[d - 1],
                recv_sem=recv_sems.at[d - 1],
                device_id=(tgt,),
                device_id_type=pl.DeviceIdType.MESH,
            )
            rdma.start()
            rdmas.append(rdma)
        out_ref[:, :, pl.ds(my * C, C)] = o_ref[...]
        for d in range(1, N_DEV):
            rdmas[d - 1].wait_recv()
            sender = lax.rem(my - d + N_DEV, N_DEV)
            out_ref[:, :, pl.ds(sender * C, C)] = recv_ref[d - 1]
        for d in range(1, N_DEV):
            rdmas[d - 1].wait_send()

    return pl.pallas_call(
        body,
        out_shape=jax.ShapeDtypeStruct((B, S, N_DEV * C), BF16),
        in_specs=[pl.BlockSpec(memory_space=pltpu.VMEM)],
        out_specs=pl.BlockSpec(memory_space=pltpu.VMEM),
        scratch_shapes=[
            pltpu.VMEM((N_DEV - 1, B, S, C), BF16),
            pltpu.SemaphoreType.DMA((N_DEV - 1,)),
            pltpu.SemaphoreType.DMA((N_DEV - 1,)),
        ],
        compiler_params=pltpu.CompilerParams(collective_id=1),
    )(o_own)


def kernel(x, Wdkv, Wuk, Wuv, Wq, Wqr, Wkr, Wo):
    B, S, D = x.shape
    H, Dh, Dr = 32, 128, 64
    hpd = H // N_DEV
    C = hpd * Dh

    my = lax.axis_index("i")
    xb = x.astype(BF16)

    c = xb @ Wdkv.astype(BF16)
    Kp = c @ Wuk.astype(BF16)
    Vp = c @ Wuv.astype(BF16)

    Kc = Kp.reshape(B, S, N_DEV, C).transpose(2, 0, 1, 3)
    Vc = Vp.reshape(B, S, N_DEV, C).transpose(2, 0, 1, 3)
    kv_chunks = jnp.stack([Kc, Vc], axis=1)
    own_chunk = lax.dynamic_index_in_dim(kv_chunks, my, axis=0, keepdims=False)

    kv_own = _a2a_reduce(kv_chunks, own_chunk)
    K = kv_own[0].reshape(B, S, hpd, Dh)
    V = kv_own[1].reshape(B, S, hpd, Dh)

    Wq_own = lax.dynamic_slice_in_dim(Wq.astype(BF16), my * C, C, axis=1)
    Q = (xb @ Wq_own).reshape(B, S, hpd, Dh)
    Wqr_own = lax.dynamic_slice_in_dim(
        Wqr.astype(BF16), my * hpd * Dr, hpd * Dr, axis=1)
    Qr = (xb @ Wqr_own).reshape(B, S, hpd, Dr)
    Kr = xb @ Wkr.astype(BF16)

    scale = (Dh + Dr) ** -0.5
    s1 = jnp.einsum("bshd,bthd->bhst", Q, K,
                    preferred_element_type=jnp.float32)
    s2 = jnp.einsum("bshd,btd->bhst", Qr, Kr,
                    preferred_element_type=jnp.float32)
    scores = (s1 + s2) * scale
    m = jnp.max(scores, axis=-1, keepdims=True)
    P = jnp.exp(scores - m)
    P = P / jnp.sum(P, axis=-1, keepdims=True)
    O = jnp.einsum("bhst,bthd->bshd", P.astype(BF16), V)
    O = O.reshape(B, S, C)

    O_full = _allgather_o(O)
    out = jnp.einsum("bsk,kd->bsd", O_full, Wo.astype(BF16),
                     preferred_element_type=jnp.float32)
    return out


# baseline (device time: 413162 ns/iter reference)
import functools

import jax
import jax.numpy as jnp
from jax import lax
from jax.experimental import pallas as pl
from jax.experimental.pallas import tpu as pltpu

N_DEV = 16
BF16 = jnp.bfloat16


def _barrier(my):
    bsem = pltpu.get_barrier_semaphore()
    for j in range(N_DEV):
        @pl.when(my != j)
        def _(j=j):
            pl.semaphore_signal(
                bsem, inc=1, device_id=(j,),
                device_id_type=pl.DeviceIdType.MESH,
            )
    pl.semaphore_wait(bsem, N_DEV - 1)


def _a2a_reduce(kv_chunks, own_chunk):
    n, two, B, S, C = kv_chunks.shape

    def body(kv_ref, own_ref, out_ref, recv_ref, send_sems, recv_sems):
        my = lax.axis_index("i")
        _barrier(my)
        rdmas = []
        for d in range(1, N_DEV):
            tgt = lax.rem(my + d, N_DEV)
            rdma = pltpu.make_async_remote_copy(
                src_ref=kv_ref.at[tgt],
                dst_ref=recv_ref.at[d - 1],
                send_sem=send_sems.at[d - 1],
                recv_sem=recv_sems.at[d - 1],
                device_id=(tgt,),
                device_id_type=pl.DeviceIdType.MESH,
            )
            rdma.start()
            rdmas.append(rdma)
        acc = own_ref[...].astype(jnp.float32)
        for d in range(1, N_DEV):
            rdmas[d - 1].wait_recv()
            acc = acc + recv_ref[d - 1].astype(jnp.float32)
        for d in range(1, N_DEV):
            rdmas[d - 1].wait_send()
        out_ref[...] = acc.astype(BF16)

    return pl.pallas_call(
        body,
        out_shape=jax.ShapeDtypeStruct((two, B, S, C), BF16),
        in_specs=[
            pl.BlockSpec(memory_space=pl.ANY),
            pl.BlockSpec(memory_space=pltpu.VMEM),
        ],
        out_specs=pl.BlockSpec(memory_space=pltpu.VMEM),
        scratch_shapes=[
            pltpu.VMEM((N_DEV - 1, two, B, S, C), BF16),
            pltpu.SemaphoreType.DMA((N_DEV - 1,)),
            pltpu.SemaphoreType.DMA((N_DEV - 1,)),
        ],
        compiler_params=pltpu.CompilerParams(collective_id=0),
    )(kv_chunks, own_chunk)


def _allgather_o(o_own):
    B, S, C = o_own.shape

    def body(o_ref, out_ref, recv_ref, send_sems, recv_sems):
        my = lax.axis_index("i")
        _barrier(my)
        rdmas = []
        for d in range(1, N_DEV):
            tgt = lax.rem(my + d, N_DEV)
            rdma = pltpu.make_async_remote_copy(
                src_ref=o_ref,
                dst_ref=recv_ref.at[d - 1],
                send_sem=send_sems.at[d - 1],
                recv_sem=recv_sems.at[d - 1],
                device_id=(tgt,),
                device_id_type=pl.DeviceIdType.MESH,
            )
            rdma.start()
            rdmas.append(rdma)
        out_ref[:, :, pl.ds(my * C, C)] = o_ref[...]
        for d in range(1, N_DEV):
            rdmas[d - 1].wait_recv()
            sender = lax.rem(my - d + N_DEV, N_DEV)
            out_ref[:, :, pl.ds(sender * C, C)] = recv_ref[d - 1]
        for d in range(1, N_DEV):
            rdmas[d - 1].wait_send()

    return pl.pallas_call(
        body,
        out_shape=jax.ShapeDtypeStruct((B, S, N_DEV * C), BF16),
        in_specs=[pl.BlockSpec(memory_space=pltpu.VMEM)],
        out_specs=pl.BlockSpec(memory_space=pltpu.VMEM),
        scratch_shapes=[
            pltpu.VMEM((N_DEV - 1, B, S, C), BF16),
            pltpu.SemaphoreType.DMA((N_DEV - 1,)),
            pltpu.SemaphoreType.DMA((N_DEV - 1,)),
        ],
        compiler_params=pltpu.CompilerParams(collective_id=1),
    )(o_own)


def kernel(x, Wdkv, Wuk, Wuv, Wq, Wqr, Wkr, Wo):
    B, S, D = x.shape
    H, Dh, Dr = 32, 128, 64
    hpd = H // N_DEV
    C = hpd * Dh

    my = lax.axis_index("i")
    xb = x.astype(BF16)

    c = xb @ Wdkv.astype(BF16)
    Kp = c @ Wuk.astype(BF16)
    Vp = c @ Wuv.astype(BF16)

    Kc = Kp.reshape(B, S, N_DEV, C).transpose(2, 0, 1, 3)
    Vc = Vp.reshape(B, S, N_DEV, C).transpose(2, 0, 1, 3)
    kv_chunks = jnp.stack([Kc, Vc], axis=1)
    own_chunk = lax.dynamic_index_in_dim(kv_chunks, my, axis=0, keepdims=False)

    kv_own = _a2a_reduce(kv_chunks, own_chunk)
    K = kv_own[0].reshape(B, S, hpd, Dh)
    V = kv_own[1].reshape(B, S, hpd, Dh)

    Wq_own = lax.dynamic_slice_in_dim(Wq.astype(BF16), my * C, C, axis=1)
    Q = (xb @ Wq_own).reshape(B, S, hpd, Dh)
    Wqr_own = lax.dynamic_slice_in_dim(
        Wqr.astype(BF16), my * hpd * Dr, hpd * Dr, axis=1)
    Qr = (xb @ Wqr_own).reshape(B, S, hpd, Dr)
    Kr = xb @ Wkr.astype(BF16)

    scale = (Dh + Dr) ** -0.5
    s1 = jnp.einsum("bshd,bthd->bhst", Q, K,
                    preferred_element_type=jnp.float32)
    s2 = jnp.einsum("bshd,btd->bhst", Qr, Kr,
                    preferred_element_type=jnp.float32)
    scores = (s1 + s2) * scale
    m = jnp.max(scores, axis=-1, keepdims=True)
    P = jnp.exp(scores - m)
    P = P / jnp.sum(P, axis=-1, keepdims=True)
    O = jnp.einsum("bhst,bthd->bshd", P.astype(BF16), V)
    O = O.reshape(B, S, C)

    O_full = _allgather_o(O)
    out = jnp.einsum("bsk,kd->bsd", O_full, Wo.astype(BF16),
                     preferred_element_type=jnp.float32)
    return out


# device time: 308227 ns/iter; 1.3404x vs baseline; 1.3404x over previous
import jax
import jax.numpy as jnp
from jax import lax
from jax.experimental import pallas as pl
from jax.experimental.pallas import tpu as pltpu

N_DEV = 16
BF16 = jnp.bfloat16


def _barrier(my):
    bsem = pltpu.get_barrier_semaphore()
    for j in range(N_DEV):
        @pl.when(my != j)
        def _(j=j):
            pl.semaphore_signal(
                bsem, inc=1, device_id=(j,),
                device_id_type=pl.DeviceIdType.MESH,
            )
    pl.semaphore_wait(bsem, N_DEV - 1)


def _kv_exchange(c2, wkv_chunks, wkv_own, x2, wq_own, wqr_own, wkr):
    M, dck = c2.shape
    _, two, _, C = wkv_chunks.shape
    D = x2.shape[1]
    Cr = wqr_own.shape[1]
    Dr = wkr.shape[1]

    def body(c_ref, w_ref, wo_ref, x_ref, wq_ref, wqr_ref, wkr_ref,
             kv_ref, q_ref, qr_ref, kr_ref,
             recv_c, recv_w, acc,
             c_send_sems, c_recv_sems, w_send_sems, w_recv_sems):
        my = lax.axis_index("i")
        _barrier(my)
        rdmas = []
        for d in range(1, N_DEV):
            tgt = lax.rem(my + d, N_DEV)
            rc = pltpu.make_async_remote_copy(
                src_ref=c_ref,
                dst_ref=recv_c.at[d - 1],
                send_sem=c_send_sems.at[d - 1],
                recv_sem=c_recv_sems.at[d - 1],
                device_id=(tgt,),
                device_id_type=pl.DeviceIdType.MESH,
            )
            rc.start()
            rw = pltpu.make_async_remote_copy(
                src_ref=w_ref.at[tgt],
                dst_ref=recv_w.at[d - 1],
                send_sem=w_send_sems.at[d - 1],
                recv_sem=w_recv_sems.at[d - 1],
                device_id=(tgt,),
                device_id_type=pl.DeviceIdType.MESH,
            )
            rw.start()
            rdmas.append((rc, rw))

        xv = x_ref[...]
        q_ref[...] = jnp.dot(
            xv, wq_ref[...], preferred_element_type=jnp.float32
        ).astype(BF16)
        qr_ref[...] = jnp.dot(
            xv, wqr_ref[...], preferred_element_type=jnp.float32
        ).astype(BF16)
        kr_ref[...] = jnp.dot(
            xv, wkr_ref[...], preferred_element_type=jnp.float32
        ).astype(BF16)

        cv = c_ref[...]
        acc[0] = jnp.dot(cv, wo_ref[0], preferred_element_type=jnp.float32)
        acc[1] = jnp.dot(cv, wo_ref[1], preferred_element_type=jnp.float32)

        for d in range(1, N_DEV):
            rc, rw = rdmas[d - 1]
            rc.wait_recv()
            rw.wait_recv()
            cr = recv_c[d - 1]
            acc[0] += jnp.dot(
                cr, recv_w[d - 1, 0], preferred_element_type=jnp.float32)
            acc[1] += jnp.dot(
                cr, recv_w[d - 1, 1], preferred_element_type=jnp.float32)
        kv_ref[...] = acc[...].astype(BF16)
        for rc, rw in rdmas:
            rc.wait_send()
            rw.wait_send()

    return pl.pallas_call(
        body,
        out_shape=(
            jax.ShapeDtypeStruct((two, M, C), BF16),
            jax.ShapeDtypeStruct((M, C), BF16),
            jax.ShapeDtypeStruct((M, Cr), BF16),
            jax.ShapeDtypeStruct((M, Dr), BF16),
        ),
        in_specs=[pl.BlockSpec(memory_space=pltpu.VMEM)] * 7,
        out_specs=(pl.BlockSpec(memory_space=pltpu.VMEM),) * 4,
        scratch_shapes=[
            pltpu.VMEM((N_DEV - 1, M, dck), BF16),
            pltpu.VMEM((N_DEV - 1, two, dck, C), BF16),
            pltpu.VMEM((two, M, C), jnp.float32),
            pltpu.SemaphoreType.DMA((N_DEV - 1,)),
            pltpu.SemaphoreType.DMA((N_DEV - 1,)),
            pltpu.SemaphoreType.DMA((N_DEV - 1,)),
            pltpu.SemaphoreType.DMA((N_DEV - 1,)),
        ],
        compiler_params=pltpu.CompilerParams(
            collective_id=0, vmem_limit_bytes=100 * 1024 * 1024),
    )(c2, wkv_chunks, wkv_own, x2, wq_own, wqr_own, wkr)


def _gather_o_matmul(o_own, wo):
    M, C = o_own.shape
    D = wo.shape[1]

    def body(o_ref, wo_ref, out_ref, o_full, send_sems, recv_sems):
        my = lax.axis_index("i")
        _barrier(my)
        rdmas = []
        for d in range(1, N_DEV):
            tgt = lax.rem(my + d, N_DEV)
            rdma = pltpu.make_async_remote_copy(
                src_ref=o_ref,
                dst_ref=o_full.at[:, pl.ds(my * C, C)],
                send_sem=send_sems.at[d - 1],
                recv_sem=recv_sems.at[d - 1],
                device_id=(tgt,),
                device_id_type=pl.DeviceIdType.MESH,
            )
            rdma.start()
            rdmas.append(rdma)
        o_full[:, pl.ds(my * C, C)] = o_ref[...]
        for rdma in rdmas:
            rdma.wait_recv()
        out_ref[...] = jnp.dot(
            o_full[...], wo_ref[...],
            preferred_element_type=jnp.float32).astype(BF16)
        for rdma in rdmas:
            rdma.wait_send()

    return pl.pallas_call(
        body,
        out_shape=jax.ShapeDtypeStruct((M, D), BF16),
        in_specs=[pl.BlockSpec(memory_space=pltpu.VMEM)] * 2,
        out_specs=pl.BlockSpec(memory_space=pltpu.VMEM),
        scratch_shapes=[
            pltpu.VMEM((M, N_DEV * C), BF16),
            pltpu.SemaphoreType.DMA((N_DEV - 1,)),
            pltpu.SemaphoreType.DMA((N_DEV - 1,)),
        ],
        compiler_params=pltpu.CompilerParams(
            collective_id=1, vmem_limit_bytes=100 * 1024 * 1024),
    )(o_own, wo)


def kernel(x, Wdkv, Wuk, Wuv, Wq, Wqr, Wkr, Wo):
    B, S, D = x.shape
    H, Dh, Dr = 32, 128, 64
    hpd = H // N_DEV
    C = hpd * Dh
    Cr = hpd * Dr
    M = B * S

    my = lax.axis_index("i")
    xb = x.astype(BF16)
    x2 = xb.reshape(M, D)

    c2 = x2 @ Wdkv.astype(BF16)

    wkv = jnp.stack([Wuk.astype(BF16), Wuv.astype(BF16)])
    wkv_chunks = wkv.reshape(2, 128, N_DEV, C).transpose(2, 0, 1, 3)
    wkv_own = lax.dynamic_index_in_dim(wkv_chunks, my, axis=0, keepdims=False)

    wq_own = lax.dynamic_slice_in_dim(Wq.astype(BF16), my * C, C, axis=1)
    wqr_own = lax.dynamic_slice_in_dim(Wqr.astype(BF16), my * Cr, Cr, axis=1)

    kv_own, q, qr, kr = _kv_exchange(
        c2, wkv_chunks, wkv_own, x2, wq_own, wqr_own, Wkr.astype(BF16))

    K = kv_own[0].reshape(B, S, hpd, Dh)
    V = kv_own[1].reshape(B, S, hpd, Dh)
    Q = q.reshape(B, S, hpd, Dh)
    Qr = qr.reshape(B, S, hpd, Dr)
    Kr = kr.reshape(B, S, Dr)

    scale = (Dh + Dr) ** -0.5
    s1 = jnp.einsum("bshd,bthd->bhst", Q, K,
                    preferred_element_type=jnp.float32)
    s2 = jnp.einsum("bshd,btd->bhst", Qr, Kr,
                    preferred_element_type=jnp.float32)
    scores = (s1 + s2) * scale
    m = jnp.max(scores, axis=-1, keepdims=True)
    P = jnp.exp(scores - m)
    P = P / jnp.sum(P, axis=-1, keepdims=True)
    O = jnp.einsum("bhst,bthd->bshd", P.astype(BF16), V)
    O = O.reshape(M, C)

    out = _gather_o_matmul(O, Wo.astype(BF16))
    return out.reshape(B, S, D).astype(jnp.float32)


# device time: 295576 ns/iter; 1.3978x vs baseline; 1.0428x over previous
import jax
import jax.numpy as jnp
from jax import lax
from jax.experimental import pallas as pl
from jax.experimental.pallas import tpu as pltpu

N_DEV = 16
BF16 = jnp.bfloat16


def _barrier(my):
    bsem = pltpu.get_barrier_semaphore()
    for j in range(N_DEV):
        @pl.when(my != j)
        def _(j=j):
            pl.semaphore_signal(
                bsem, inc=1, device_id=(j,),
                device_id_type=pl.DeviceIdType.MESH,
            )
    pl.semaphore_wait(bsem, N_DEV - 1)


def _kv_exchange(c2, wkv_chunks, wkv_own, x2, wq_own, wqr_own, wkr):
    M, dck = c2.shape
    _, two, _, C = wkv_chunks.shape
    D = x2.shape[1]
    Cr = wqr_own.shape[1]
    Dr = wkr.shape[1]

    def body(c_ref, w_ref, wo_ref, x_ref, wq_ref, wqr_ref, wkr_ref,
             kv_ref, q_ref, qr_ref, kr_ref,
             recv_c, recv_w, acc,
             c_send_sems, c_recv_sems, w_send_sems, w_recv_sems):
        my = lax.axis_index("i")
        _barrier(my)
        rdmas = []
        for d in range(1, N_DEV):
            tgt = lax.rem(my + d, N_DEV)
            rc = pltpu.make_async_remote_copy(
                src_ref=c_ref,
                dst_ref=recv_c.at[d - 1],
                send_sem=c_send_sems.at[d - 1],
                recv_sem=c_recv_sems.at[d - 1],
                device_id=(tgt,),
                device_id_type=pl.DeviceIdType.MESH,
            )
            rc.start()
            rw = pltpu.make_async_remote_copy(
                src_ref=w_ref.at[tgt],
                dst_ref=recv_w.at[d - 1],
                send_sem=w_send_sems.at[d - 1],
                recv_sem=w_recv_sems.at[d - 1],
                device_id=(tgt,),
                device_id_type=pl.DeviceIdType.MESH,
            )
            rw.start()
            rdmas.append((rc, rw))

        xv = x_ref[...]
        q_ref[...] = jnp.dot(
            xv, wq_ref[...], preferred_element_type=jnp.float32
        ).astype(BF16)
        qr_ref[...] = jnp.dot(
            xv, wqr_ref[...], preferred_element_type=jnp.float32
        ).astype(BF16)
        kr_ref[...] = jnp.dot(
            xv, wkr_ref[...], preferred_element_type=jnp.float32
        ).astype(BF16)

        cv = c_ref[...]
        acc[0] = jnp.dot(cv, wo_ref[0], preferred_element_type=jnp.float32)
        acc[1] = jnp.dot(cv, wo_ref[1], preferred_element_type=jnp.float32)

        for d in range(1, N_DEV):
            rc, rw = rdmas[d - 1]
            rc.wait_recv()
            rw.wait_recv()
            cr = recv_c[d - 1]
            acc[0] += jnp.dot(
                cr, recv_w[d - 1, 0], preferred_element_type=jnp.float32)
            acc[1] += jnp.dot(
                cr, recv_w[d - 1, 1], preferred_element_type=jnp.float32)
        kv_ref[...] = acc[...].astype(BF16)
        for rc, rw in rdmas:
            rc.wait_send()
            rw.wait_send()

    return pl.pallas_call(
        body,
        out_shape=(
            jax.ShapeDtypeStruct((two, M, C), BF16),
            jax.ShapeDtypeStruct((M, C), BF16),
            jax.ShapeDtypeStruct((M, Cr), BF16),
            jax.ShapeDtypeStruct((M, Dr), BF16),
        ),
        in_specs=[pl.BlockSpec(memory_space=pltpu.VMEM)] * 7,
        out_specs=(pl.BlockSpec(memory_space=pltpu.VMEM),) * 4,
        scratch_shapes=[
            pltpu.VMEM((N_DEV - 1, M, dck), BF16),
            pltpu.VMEM((N_DEV - 1, two, dck, C), BF16),
            pltpu.VMEM((two, M, C), jnp.float32),
            pltpu.SemaphoreType.DMA((N_DEV - 1,)),
            pltpu.SemaphoreType.DMA((N_DEV - 1,)),
            pltpu.SemaphoreType.DMA((N_DEV - 1,)),
            pltpu.SemaphoreType.DMA((N_DEV - 1,)),
        ],
        compiler_params=pltpu.CompilerParams(
            collective_id=0, vmem_limit_bytes=100 * 1024 * 1024),
    )(c2, wkv_chunks, wkv_own, x2, wq_own, wqr_own, wkr)


def _gather_o_matmul(o_own, wo_rot):
    M, C = o_own.shape
    D = wo_rot.shape[1]
    GC = 4 * C

    def body(o_ref, wo_ref, out_ref, o_rel, wstage, send_sems, recv_sems,
             dma_sems):
        my = lax.axis_index("i")
        _barrier(my)
        recv_by_slot = {}
        rdmas = []
        for d in range(1, N_DEV):
            tgt = lax.rem(my + d, N_DEV)
            s = N_DEV - d
            rdma = pltpu.make_async_remote_copy(
                src_ref=o_ref,
                dst_ref=o_rel.at[:, pl.ds(s * C, C)],
                send_sem=send_sems.at[d - 1],
                recv_sem=recv_sems.at[s - 1],
                device_id=(tgt,),
                device_id_type=pl.DeviceIdType.MESH,
            )
            rdma.start()
            rdmas.append(rdma)
            recv_by_slot[s] = rdma
        o_rel[:, 0:C] = o_ref[...]

        order = (0, 3, 1, 2)
        wdmas = {}
        for i, g in enumerate(order):
            buf = i % 2
            wdma = pltpu.make_async_copy(
                wo_ref.at[pl.ds(g * GC, GC), :], wstage.at[buf],
                dma_sems.at[buf])
            if i < 2:
                wdma.start()
            wdmas[g] = (buf, wdma)
        for i, g in enumerate(order):
            for s in range(g * 4, g * 4 + 4):
                if s > 0:
                    recv_by_slot[s].wait_recv()
            buf, wdma = wdmas[g]
            wdma.wait()
            partial = jnp.dot(
                o_rel[:, pl.ds(g * GC, GC)], wstage[buf],
                preferred_element_type=jnp.float32)
            if i == 0:
                out_ref[...] = partial
            else:
                out_ref[...] += partial
            if i + 2 < len(order):
                nbuf, nwdma = wdmas[order[i + 2]]
                nwdma.start()
        for rdma in rdmas:
            rdma.wait_send()

    return pl.pallas_call(
        body,
        out_shape=jax.ShapeDtypeStruct((M, D), jnp.float32),
        in_specs=[
            pl.BlockSpec(memory_space=pltpu.VMEM),
            pl.BlockSpec(memory_space=pl.ANY),
        ],
        out_specs=pl.BlockSpec(memory_space=pltpu.VMEM),
        scratch_shapes=[
            pltpu.VMEM((M, N_DEV * C), BF16),
            pltpu.VMEM((2, GC, D), BF16),
            pltpu.SemaphoreType.DMA((N_DEV - 1,)),
            pltpu.SemaphoreType.DMA((N_DEV - 1,)),
            pltpu.SemaphoreType.DMA((2,)),
        ],
        compiler_params=pltpu.CompilerParams(
            collective_id=1, vmem_limit_bytes=100 * 1024 * 1024),
    )(o_own, wo_rot)


def kernel(x, Wdkv, Wuk, Wuv, Wq, Wqr, Wkr, Wo):
    B, S, D = x.shape
    H, Dh, Dr = 32, 128, 64
    hpd = H // N_DEV
    C = hpd * Dh
    Cr = hpd * Dr
    M = B * S

    my = lax.axis_index("i")
    xb = x.astype(BF16)
    x2 = xb.reshape(M, D)

    c2 = x2 @ Wdkv.astype(BF16)

    wkv = jnp.stack([Wuk.astype(BF16), Wuv.astype(BF16)])
    wkv_chunks = wkv.reshape(2, 128, N_DEV, C).transpose(2, 0, 1, 3)
    wkv_own = lax.dynamic_index_in_dim(wkv_chunks, my, axis=0, keepdims=False)

    wq_own = lax.dynamic_slice_in_dim(Wq.astype(BF16), my * C, C, axis=1)
    wqr_own = lax.dynamic_slice_in_dim(Wqr.astype(BF16), my * Cr, Cr, axis=1)

    kv_own, q, qr, kr = _kv_exchange(
        c2, wkv_chunks, wkv_own, x2, wq_own, wqr_own, Wkr.astype(BF16))

    K = kv_own[0].reshape(B, S, hpd, Dh)
    V = kv_own[1].reshape(B, S, hpd, Dh)
    Q = q.reshape(B, S, hpd, Dh)
    Qr = qr.reshape(B, S, hpd, Dr)
    Kr = kr.reshape(B, S, Dr)

    scale = (Dh + Dr) ** -0.5
    s1 = jnp.einsum("bshd,bthd->bhst", Q, K,
                    preferred_element_type=jnp.float32)
    s2 = jnp.einsum("bshd,btd->bhst", Qr, Kr,
                    preferred_element_type=jnp.float32)
    scores = (s1 + s2) * scale
    m = jnp.max(scores, axis=-1, keepdims=True)
    P = jnp.exp(scores - m)
    P = P / jnp.sum(P, axis=-1, keepdims=True)
    O = jnp.einsum("bhst,bthd->bshd", P.astype(BF16), V)
    O = O.reshape(M, C)

    wo_rot = jnp.roll(Wo, -my * C, axis=0).astype(BF16)
    out = _gather_o_matmul(O, wo_rot)
    return out.reshape(B, S, D)


# device time: 238893 ns/iter; 1.7295x vs baseline; 1.2373x over previous
import jax
import jax.numpy as jnp
from jax import lax
from jax.experimental import pallas as pl
from jax.experimental.pallas import tpu as pltpu

N_DEV = 16
BF16 = jnp.bfloat16


def _barrier(my):
    bsem = pltpu.get_barrier_semaphore()
    for j in range(N_DEV):
        @pl.when(my != j)
        def _(j=j):
            pl.semaphore_signal(
                bsem, inc=1, device_id=(j,),
                device_id_type=pl.DeviceIdType.MESH,
            )
    pl.semaphore_wait(bsem, N_DEV - 1)


def _kv_exchange(c2, wkv_chunks, wkv_own, x2, wq, wqr, wkr, wo):
    M, dck = c2.shape
    _, two, _, C = wkv_chunks.shape
    D = x2.shape[1]
    Cr = wqr.shape[1] // N_DEV
    Dr = wkr.shape[1]

    def body(c_ref, w_ref, wo_own_ref, x_ref, wq_ref, wqr_ref, wkr_ref,
             wo_ref,
             kv_ref, q_ref, qr_ref, kr_ref, wrot_ref,
             recv_c, recv_w, acc, qf, qrf, wf, wb,
             c_send_sems, c_recv_sems, w_send_sems, w_recv_sems,
             q_sems, win_sems, wout_sems):
        my = lax.axis_index("i")
        _barrier(my)
        rdmas = []
        for d in range(1, N_DEV):
            tgt = lax.rem(my + d, N_DEV)
            rc = pltpu.make_async_remote_copy(
                src_ref=c_ref,
                dst_ref=recv_c.at[d - 1],
                send_sem=c_send_sems.at[d - 1],
                recv_sem=c_recv_sems.at[d - 1],
                device_id=(tgt,),
                device_id_type=pl.DeviceIdType.MESH,
            )
            rc.start()
            rw = pltpu.make_async_remote_copy(
                src_ref=w_ref.at[tgt],
                dst_ref=recv_w.at[d - 1],
                send_sem=w_send_sems.at[d - 1],
                recv_sem=w_recv_sems.at[d - 1],
                device_id=(tgt,),
                device_id_type=pl.DeviceIdType.MESH,
            )
            rw.start()
            rdmas.append((rc, rw))

        qdma = pltpu.make_async_copy(
            wq_ref.at[:, pl.ds(my * C, C)], qf, q_sems.at[0])
        qdma.start()
        qrdma = pltpu.make_async_copy(
            wqr_ref.at[:, pl.ds(my * Cr, Cr)], qrf, q_sems.at[1])
        qrdma.start()

        def win(j):
            start = lax.rem(my + j, N_DEV) * C
            return pltpu.make_async_copy(
                wo_ref.at[pl.ds(start, C), :], wf.at[j % 2],
                win_sems.at[j % 2])
        def wout(j):
            return pltpu.make_async_copy(
                wb.at[j % 2], wrot_ref.at[pl.ds(j * C, C), :],
                wout_sems.at[j % 2])
        win(0).start()
        win(1).start()
        for j in range(N_DEV):
            win(j).wait()
            if j >= 2:
                wout(j - 2).wait()
            wb[j % 2] = wf[j % 2].astype(BF16)
            wout(j).start()
            if j + 2 < N_DEV:
                win(j + 2).start()

        xv = x_ref[...]
        qdma.wait()
        q_ref[...] = jnp.dot(
            xv, qf[...].astype(BF16), preferred_element_type=jnp.float32
        ).astype(BF16)
        qrdma.wait()
        qr_ref[...] = jnp.dot(
            xv, qrf[...].astype(BF16), preferred_element_type=jnp.float32
        ).astype(BF16)
        kr_ref[...] = jnp.dot(
            xv, wkr_ref[...], preferred_element_type=jnp.float32
        ).astype(BF16)

        cv = c_ref[...]
        acc[0] = jnp.dot(cv, wo_own_ref[0], preferred_element_type=jnp.float32)
        acc[1] = jnp.dot(cv, wo_own_ref[1], preferred_element_type=jnp.float32)

        for d in range(1, N_DEV):
            rc, rw = rdmas[d - 1]
            rc.wait_recv()
            rw.wait_recv()
            cr = recv_c[d - 1]
            acc[0] += jnp.dot(
                cr, recv_w[d - 1, 0], preferred_element_type=jnp.float32)
            acc[1] += jnp.dot(
                cr, recv_w[d - 1, 1], preferred_element_type=jnp.float32)
        kv_ref[...] = acc[...].astype(BF16)
        wout(N_DEV - 2).wait()
        wout(N_DEV - 1).wait()
        for rc, rw in rdmas:
            rc.wait_send()
            rw.wait_send()

    return pl.pallas_call(
        body,
        out_shape=(
            jax.ShapeDtypeStruct((two, M, C), BF16),
            jax.ShapeDtypeStruct((M, C), BF16),
            jax.ShapeDtypeStruct((M, Cr), BF16),
            jax.ShapeDtypeStruct((M, Dr), BF16),
            jax.ShapeDtypeStruct((D, D), BF16),
        ),
        in_specs=[pl.BlockSpec(memory_space=pltpu.VMEM)] * 4
        + [pl.BlockSpec(memory_space=pl.ANY)] * 2
        + [pl.BlockSpec(memory_space=pltpu.VMEM)]
        + [pl.BlockSpec(memory_space=pl.ANY)],
        out_specs=(pl.BlockSpec(memory_space=pltpu.VMEM),) * 4
        + (pl.BlockSpec(memory_space=pl.ANY),),
        scratch_shapes=[
            pltpu.VMEM((N_DEV - 1, M, dck), BF16),
            pltpu.VMEM((N_DEV - 1, two, dck, C), BF16),
            pltpu.VMEM((two, M, C), jnp.float32),
            pltpu.VMEM((D, C), jnp.float32),
            pltpu.VMEM((D, Cr), jnp.float32),
            pltpu.VMEM((2, C, D), jnp.float32),
            pltpu.VMEM((2, C, D), BF16),
            pltpu.SemaphoreType.DMA((N_DEV - 1,)),
            pltpu.SemaphoreType.DMA((N_DEV - 1,)),
            pltpu.SemaphoreType.DMA((N_DEV - 1,)),
            pltpu.SemaphoreType.DMA((N_DEV - 1,)),
            pltpu.SemaphoreType.DMA((2,)),
            pltpu.SemaphoreType.DMA((2,)),
            pltpu.SemaphoreType.DMA((2,)),
        ],
        compiler_params=pltpu.CompilerParams(
            collective_id=0, vmem_limit_bytes=100 * 1024 * 1024),
    )(c2, wkv_chunks, wkv_own, x2, wq, wqr, wkr, wo)


def _gather_o_matmul(o_own, wo_rot):
    M, C = o_own.shape
    D = wo_rot.shape[1]
    GC = 4 * C

    def body(o_ref, wo_ref, out_ref, o_rel, wstage, send_sems, recv_sems,
             dma_sems):
        my = lax.axis_index("i")
        _barrier(my)
        recv_by_slot = {}
        rdmas = []
        for d in range(1, N_DEV):
            tgt = lax.rem(my + d, N_DEV)
            s = N_DEV - d
            rdma = pltpu.make_async_remote_copy(
                src_ref=o_ref,
                dst_ref=o_rel.at[:, pl.ds(s * C, C)],
                send_sem=send_sems.at[d - 1],
                recv_sem=recv_sems.at[s - 1],
                device_id=(tgt,),
                device_id_type=pl.DeviceIdType.MESH,
            )
            rdma.start()
            rdmas.append(rdma)
            recv_by_slot[s] = rdma
        o_rel[:, 0:C] = o_ref[...]

        order = (0, 3, 1, 2)
        wdmas = {}
        for i, g in enumerate(order):
            buf = i % 2
            wdma = pltpu.make_async_copy(
                wo_ref.at[pl.ds(g * GC, GC), :], wstage.at[buf],
                dma_sems.at[buf])
            if i < 2:
                wdma.start()
            wdmas[g] = (buf, wdma)
        for i, g in enumerate(order):
            for s in range(g * 4, g * 4 + 4):
                if s > 0:
                    recv_by_slot[s].wait_recv()
            buf, wdma = wdmas[g]
            wdma.wait()
            partial = jnp.dot(
                o_rel[:, pl.ds(g * GC, GC)], wstage[buf],
                preferred_element_type=jnp.float32)
            if i == 0:
                out_ref[...] = partial
            else:
                out_ref[...] += partial
            if i + 2 < len(order):
                nbuf, nwdma = wdmas[order[i + 2]]
                nwdma.start()
        for rdma in rdmas:
            rdma.wait_send()

    return pl.pallas_call(
        body,
        out_shape=jax.ShapeDtypeStruct((M, D), jnp.float32),
        in_specs=[
            pl.BlockSpec(memory_space=pltpu.VMEM),
            pl.BlockSpec(memory_space=pl.ANY),
        ],
        out_specs=pl.BlockSpec(memory_space=pltpu.VMEM),
        scratch_shapes=[
            pltpu.VMEM((M, N_DEV * C), BF16),
            pltpu.VMEM((2, GC, D), BF16),
            pltpu.SemaphoreType.DMA((N_DEV - 1,)),
            pltpu.SemaphoreType.DMA((N_DEV - 1,)),
            pltpu.SemaphoreType.DMA((2,)),
        ],
        compiler_params=pltpu.CompilerParams(
            collective_id=1, vmem_limit_bytes=100 * 1024 * 1024),
    )(o_own, wo_rot)


def kernel(x, Wdkv, Wuk, Wuv, Wq, Wqr, Wkr, Wo):
    B, S, D = x.shape
    H, Dh, Dr = 32, 128, 64
    hpd = H // N_DEV
    C = hpd * Dh
    Cr = hpd * Dr
    M = B * S

    my = lax.axis_index("i")
    xb = x.astype(BF16)
    x2 = xb.reshape(M, D)

    c2 = x2 @ Wdkv.astype(BF16)

    wkv = jnp.stack([Wuk.astype(BF16), Wuv.astype(BF16)])
    wkv_chunks = wkv.reshape(2, 128, N_DEV, C).transpose(2, 0, 1, 3)
    wkv_own = lax.dynamic_index_in_dim(wkv_chunks, my, axis=0, keepdims=False)

    kv_own, q, qr, kr, wo_rot = _kv_exchange(
        c2, wkv_chunks, wkv_own, x2, Wq, Wqr, Wkr.astype(BF16), Wo)

    K = kv_own[0].reshape(B, S, hpd, Dh)
    V = kv_own[1].reshape(B, S, hpd, Dh)
    Q = q.reshape(B, S, hpd, Dh)
    Qr = qr.reshape(B, S, hpd, Dr)
    Kr = kr.reshape(B, S, Dr)

    scale = (Dh + Dr) ** -0.5
    s1 = jnp.einsum("bshd,bthd->bhst", Q, K,
                    preferred_element_type=jnp.float32)
    s2 = jnp.einsum("bshd,btd->bhst", Qr, Kr,
                    preferred_element_type=jnp.float32)
    scores = (s1 + s2) * scale
    m = jnp.max(scores, axis=-1, keepdims=True)
    P = jnp.exp(scores - m)
    P = P / jnp.sum(P, axis=-1, keepdims=True)
    O = jnp.einsum("bhst,bthd->bshd", P.astype(BF16), V)
    O = O.reshape(M, C)

    out = _gather_o_matmul(O, wo_rot)
    return out.reshape(B, S, D)
